# baseline (speedup 1.0000x reference)
import functools

import jax
import jax.numpy as jnp
from jax import lax
from jax.experimental import pallas as pl
from jax.experimental.pallas import tpu as pltpu
from jax.experimental.pallas import tpu_sc as plsc

_NBUF = 2


@functools.lru_cache(maxsize=None)
def _build_gather(B, V, D, C):
    info = plsc.get_sparse_core_info()
    NC, NS = info.num_cores, info.num_subcores
    NW = NC * NS
    assert B % NW == 0
    b_per_w = B // NW
    assert b_per_w % (C * _NBUF) == 0
    n_chunks = b_per_w // C

    mesh = plsc.VectorSubcoreMesh(core_axis_name="c", subcore_axis_name="s")

    @functools.partial(
        pl.kernel,
        mesh=mesh,
        out_type=jax.ShapeDtypeStruct((B, D), jnp.float32),
        scratch_types=[
            pltpu.VMEM((_NBUF, C), jnp.int32),
            pltpu.VMEM((_NBUF, C, D), jnp.float32),
            pltpu.SemaphoreType.DMA((_NBUF,)),
            pltpu.SemaphoreType.DMA((_NBUF,)),
            pltpu.SemaphoreType.DMA((_NBUF,)),
        ],
        compiler_params=pltpu.CompilerParams(use_tc_tiling_on_sc=False),
    )
    def gather_kernel(idx_hbm, table_hbm, out_hbm, idx_v, rows_v, isem, gsem, osem):
        wid = lax.axis_index("s") * NC + lax.axis_index("c")
        base = wid * b_per_w

        for b in range(_NBUF):
            pltpu.async_copy(
                idx_hbm.at[pl.ds(base + b * C, C)], idx_v.at[b], isem.at[b]
            )

        def body(j, carry):
            for b in range(_NBUF):
                i = j * _NBUF + b
                off = base + i * C
                pltpu.make_async_copy(
                    idx_hbm.at[pl.ds(off, C)], idx_v.at[b], isem.at[b]
                ).wait()
                @pl.when(j > 0)
                def _():
                    pltpu.make_async_copy(
                        rows_v.at[b], out_hbm.at[pl.ds(off, C)], osem.at[b]
                    ).wait()

                pltpu.async_copy(
                    table_hbm.at[idx_v.at[b]], rows_v.at[b], gsem.at[b]
                ).wait()
                pltpu.async_copy(rows_v.at[b], out_hbm.at[pl.ds(off, C)], osem.at[b])
                @pl.when(i + _NBUF < n_chunks)
                def _():
                    pltpu.async_copy(
                        idx_hbm.at[pl.ds(off + _NBUF * C, C)],
                        idx_v.at[b],
                        isem.at[b],
                    )

            return carry

        lax.fori_loop(0, n_chunks // _NBUF, body, 0)

        for b in range(_NBUF):
            pltpu.make_async_copy(
                rows_v.at[b], out_hbm.at[pl.ds(base, C)], osem.at[b]
            ).wait()

    return gather_kernel


def kernel(inputs, embeddings):
    B = inputs.shape[0] * inputs.shape[1]
    V, D = embeddings.shape
    flat = jnp.reshape(inputs, (B,)).astype(jnp.int32)
    out = _build_gather(B, V, D, 1600)(flat, embeddings)
    return jnp.reshape(out, inputs.shape + (D,))


# speedup vs baseline: 1.0000x; 1.0000x over previous
import functools

import jax
import jax.numpy as jnp
from jax import lax
from jax.experimental import pallas as pl
from jax.experimental.pallas import tpu as pltpu
from jax.experimental.pallas import tpu_sc as plsc

_NBUF = 2
_G = 4


@functools.lru_cache(maxsize=None)
def _build_gather(B, V, D, C):
    info = plsc.get_sparse_core_info()
    NC, NS = info.num_cores, info.num_subcores
    NW = NC * NS
    assert B % NW == 0
    b_per_w = B // NW
    assert b_per_w % (C * _NBUF) == 0
    n_chunks = b_per_w // C

    mesh = plsc.VectorSubcoreMesh(core_axis_name="c", subcore_axis_name="s")

    @functools.partial(
        pl.kernel,
        mesh=mesh,
        out_type=jax.ShapeDtypeStruct((B, D), jnp.float32),
        scratch_types=[
            pltpu.VMEM((_NBUF, C), jnp.int32),
            pltpu.VMEM((_NBUF, C, D), jnp.float32),
            pltpu.SemaphoreType.DMA((_NBUF,)),
            pltpu.SemaphoreType.DMA((_NBUF,)),
            pltpu.SemaphoreType.DMA((_NBUF,)),
        ],
        compiler_params=pltpu.CompilerParams(use_tc_tiling_on_sc=False),
    )
    def gather_kernel(idx_hbm, table_hbm, out_hbm, idx_v, rows_v, isem, gsem, osem):
        wid = lax.axis_index("s") * NC + lax.axis_index("c")
        base = wid * b_per_w

        for b in range(_NBUF):
            pltpu.async_copy(
                idx_hbm.at[pl.ds(base + b * C, C)], idx_v.at[b], isem.at[b]
            )

        def body(j, carry):
            for b in range(_NBUF):
                i = j * _NBUF + b
                off = base + i * C
                pltpu.make_async_copy(
                    idx_hbm.at[pl.ds(off, C)], idx_v.at[b], isem.at[b]
                ).wait()
                @pl.when(j > 0)
                def _():
                    pltpu.make_async_copy(
                        rows_v.at[b], out_hbm.at[pl.ds(off, C)], osem.at[b]
                    ).wait()

                CG = C // _G
                for g in range(_G):
                    pltpu.async_copy(
                        table_hbm.at[idx_v.at[b, pl.ds(g * CG, CG)]],
                        rows_v.at[b, pl.ds(g * CG, CG)],
                        gsem.at[b],
                    )
                pltpu.make_async_copy(
                    table_hbm.at[idx_v.at[b]], rows_v.at[b], gsem.at[b]
                ).wait()
                pltpu.async_copy(rows_v.at[b], out_hbm.at[pl.ds(off, C)], osem.at[b])
                @pl.when(i + _NBUF < n_chunks)
                def _():
                    pltpu.async_copy(
                        idx_hbm.at[pl.ds(off + _NBUF * C, C)],
                        idx_v.at[b],
                        isem.at[b],
                    )

            return carry

        lax.fori_loop(0, n_chunks // _NBUF, body, 0)

        for b in range(_NBUF):
            pltpu.make_async_copy(
                rows_v.at[b], out_hbm.at[pl.ds(base, C)], osem.at[b]
            ).wait()

    return gather_kernel


def kernel(inputs, embeddings):
    B = inputs.shape[0] * inputs.shape[1]
    V, D = embeddings.shape
    flat = jnp.reshape(inputs, (B,)).astype(jnp.int32)
    out = _build_gather(B, V, D, 1600)(flat, embeddings)
    return jnp.reshape(out, inputs.shape + (D,))


# speedup vs baseline: 1.3697x; 1.3697x over previous
import functools

import jax
import jax.numpy as jnp
from jax import lax
from jax.experimental import pallas as pl
from jax.experimental.pallas import tpu as pltpu
from jax.experimental.pallas import tpu_sc as plsc


@functools.lru_cache(maxsize=None)
def _build_gather(S, B, V, D, C):
    info = plsc.get_sparse_core_info()
    NC, NS, L = info.num_cores, info.num_subcores, info.num_lanes
    NW = NC * NS
    assert B % (NW * C) == 0 and C % 128 == 0 and D == 32
    CB = C // 128
    DB = D // 8

    mesh = plsc.VectorSubcoreMesh(core_axis_name="c", subcore_axis_name="s")

    @functools.partial(
        pl.kernel,
        mesh=mesh,
        out_type=jax.ShapeDtypeStruct((S, DB, B // 128, 8, 128), jnp.float32),
        scratch_types=[
            pltpu.VMEM((C,), jnp.int32),
            pltpu.VMEM((C, D), jnp.float32),
            pltpu.VMEM((DB, CB, 8, 128), jnp.float32),
            pltpu.SemaphoreType.DMA,
            pltpu.SemaphoreType.DMA,
        ],
        compiler_params=pltpu.CompilerParams(
            use_tc_tiling_on_sc=False, needs_layout_passes=False
        ),
    )
    def gather_kernel(idx_hbm, table_hbm, out_hbm, idx_v, rows_v, obuf, gsem, osem):
        wid = lax.axis_index("s") * NC + lax.axis_index("c")
        b0 = wid * C
        lane = lax.iota(jnp.int32, 16)

        def body(s, carry):
            pltpu.sync_copy(idx_hbm.at[s, pl.ds(b0, C)], idx_v)
            pltpu.async_copy(table_hbm.at[idx_v], rows_v, gsem).wait()

            @pl.when(s > 0)
            def _():
                for dblk in range(DB):
                    pltpu.make_async_copy(
                        obuf.at[dblk],
                        out_hbm.at[0, dblk, pl.ds(wid * CB, CB)],
                        osem,
                    ).wait()

            def trans(t, c):
                dblk = t // (CB * 8)
                rem = t % (CB * 8)
                bblk = rem // 8
                dsub = rem % 8
                d = dblk * 8 + dsub
                dvec = jnp.full((L,), d, jnp.int32)
                for grp in range(128 // L):
                    rvec = bblk * 128 + grp * L + lane
                    vals = plsc.load_gather(rows_v, [rvec, dvec])
                    obuf[dblk, bblk, dsub, pl.ds(grp * L, L)] = vals
                return c

            lax.fori_loop(0, DB * CB * 8, trans, 0)

            for dblk in range(DB):
                pltpu.async_copy(
                    obuf.at[dblk],
                    out_hbm.at[s, dblk, pl.ds(wid * CB, CB)],
                    osem,
                )
            return carry

        lax.fori_loop(0, S, body, 0)

        for dblk in range(DB):
            pltpu.make_async_copy(
                obuf.at[dblk],
                out_hbm.at[0, dblk, pl.ds(wid * CB, CB)],
                osem,
            ).wait()

    return gather_kernel


def kernel(inputs, embeddings):
    B, S = inputs.shape
    V, D = embeddings.shape
    idx_t = jnp.transpose(inputs).astype(jnp.int32)
    out5 = _build_gather(S, B, V, D, 512)(idx_t, embeddings)
    out = jnp.transpose(out5, (2, 4, 0, 1, 3)).reshape(B, S, D)
    return out


# speedup vs baseline: 1.4431x; 1.0535x over previous
import functools

import jax
import jax.numpy as jnp
from jax import lax
from jax.experimental import pallas as pl
from jax.experimental.pallas import tpu as pltpu
from jax.experimental.pallas import tpu_sc as plsc


@functools.lru_cache(maxsize=None)
def _build_gather(S, B, V, D, C):
    info = plsc.get_sparse_core_info()
    NC, NS, L = info.num_cores, info.num_subcores, info.num_lanes
    NW = NC * NS
    assert B % (NW * C) == 0 and C % 128 == 0 and D == 32 and L == 16
    CB = C // 128
    DB = D // 8

    mesh = plsc.VectorSubcoreMesh(core_axis_name="c", subcore_axis_name="s")

    @functools.partial(
        pl.kernel,
        mesh=mesh,
        out_type=jax.ShapeDtypeStruct((S, DB, B // 128, 8, 128), jnp.float32),
        scratch_types=[
            pltpu.VMEM((2, C), jnp.int32),
            pltpu.VMEM((2, C, D), jnp.float32),
            pltpu.VMEM((DB, CB, 8, 128), jnp.float32),
            pltpu.SemaphoreType.DMA((2,)),
            pltpu.SemaphoreType.DMA,
        ],
        compiler_params=pltpu.CompilerParams(
            use_tc_tiling_on_sc=False, needs_layout_passes=False
        ),
    )
    def gather_kernel(idx_hbm, table_hbm, out_hbm, idx_v, rows_v, obuf, gsem, osem):
        wid = lax.axis_index("s") * NC + lax.axis_index("c")
        b0 = wid * C
        lane = lax.iota(jnp.int32, L)

        pltpu.sync_copy(idx_hbm.at[0, pl.ds(b0, C)], idx_v.at[0])
        pltpu.async_copy(table_hbm.at[idx_v.at[0]], rows_v.at[0], gsem.at[0])

        def body(s, carry):
            p = lax.rem(s, 2)
            q = 1 - p

            @pl.when(s + 1 < S)
            def _():
                pltpu.sync_copy(idx_hbm.at[s + 1, pl.ds(b0, C)], idx_v.at[q])
                pltpu.async_copy(
                    table_hbm.at[idx_v.at[q]], rows_v.at[q], gsem.at[q]
                )

            pltpu.make_async_copy(
                table_hbm.at[idx_v.at[p]], rows_v.at[p], gsem.at[p]
            ).wait()

            @pl.when(s > 0)
            def _():
                for dblk in range(DB):
                    pltpu.make_async_copy(
                        obuf.at[dblk],
                        out_hbm.at[0, dblk, pl.ds(wid * CB, CB)],
                        osem,
                    ).wait()

            pvec = jnp.full((L,), p, jnp.int32)

            def trans(u, c):
                dblk = u // CB
                bblk = lax.rem(u, CB)
                rbase = bblk * 128 + lane
                dbase = jnp.full((L,), dblk * 8, jnp.int32)
                for dsub in range(8):
                    dvec = dbase + dsub
                    for grp in range(128 // L):
                        rvec = rbase + grp * L
                        vals = plsc.load_gather(rows_v, [pvec, rvec, dvec])
                        obuf[dblk, bblk, dsub, pl.ds(grp * L, L)] = vals
                return c

            lax.fori_loop(0, DB * CB, trans, 0)

            for dblk in range(DB):
                pltpu.async_copy(
                    obuf.at[dblk],
                    out_hbm.at[s, dblk, pl.ds(wid * CB, CB)],
                    osem,
                )
            return carry

        lax.fori_loop(0, S, body, 0)

        for dblk in range(DB):
            pltpu.make_async_copy(
                obuf.at[dblk],
                out_hbm.at[0, dblk, pl.ds(wid * CB, CB)],
                osem,
            ).wait()

    return gather_kernel


def kernel(inputs, embeddings):
    B, S = inputs.shape
    V, D = embeddings.shape
    idx_t = jnp.transpose(inputs).astype(jnp.int32)
    out5 = _build_gather(S, B, V, D, 512)(idx_t, embeddings)
    out = jnp.transpose(out5, (2, 4, 0, 1, 3)).reshape(B, S, D)
    return out


# speedup vs baseline: 1.7270x; 1.1968x over previous
import functools

import jax
import jax.numpy as jnp
from jax import lax
from jax.experimental import pallas as pl
from jax.experimental.pallas import tpu as pltpu
from jax.experimental.pallas import tpu_sc as plsc


@functools.lru_cache(maxsize=None)
def _build_gather(S, B, V, D, C):
    info = plsc.get_sparse_core_info()
    NC, NS, L = info.num_cores, info.num_subcores, info.num_lanes
    NW = NC * NS
    assert B % (NW * C) == 0 and C % 128 == 0 and D == 32 and L == 16
    CB = C // 128
    DB = D // 8

    mesh = plsc.VectorSubcoreMesh(core_axis_name="c", subcore_axis_name="s")

    @functools.partial(
        pl.kernel,
        mesh=mesh,
        out_type=jax.ShapeDtypeStruct((S, DB, B // 128, 8, 128), jnp.float32),
        scratch_types=[
            pltpu.VMEM((2, C), jnp.int32),
            pltpu.VMEM((2 * C, D), jnp.float32),
            pltpu.VMEM((DB, CB, 8, 128), jnp.float32),
            pltpu.SemaphoreType.DMA((2,)),
            pltpu.SemaphoreType.DMA,
        ],
        compiler_params=pltpu.CompilerParams(
            use_tc_tiling_on_sc=False, needs_layout_passes=False
        ),
    )
    def gather_kernel(idx_hbm, table_hbm, out_hbm, idx_v, rows_v, obuf, gsem, osem):
        wid = lax.axis_index("s") * NC + lax.axis_index("c")
        b0 = wid * C
        lane = lax.iota(jnp.int32, L)

        pltpu.sync_copy(idx_hbm.at[0, pl.ds(b0, C)], idx_v.at[0])
        pltpu.async_copy(
            table_hbm.at[idx_v.at[0]], rows_v.at[pl.ds(0, C)], gsem.at[0]
        )

        def body(s, carry):
            p = lax.rem(s, 2)
            q = 1 - p

            @pl.when(s + 1 < S)
            def _():
                pltpu.sync_copy(idx_hbm.at[s + 1, pl.ds(b0, C)], idx_v.at[q])
                pltpu.async_copy(
                    table_hbm.at[idx_v.at[q]], rows_v.at[pl.ds(q * C, C)], gsem.at[q]
                )

            pltpu.make_async_copy(
                table_hbm.at[idx_v.at[p]], rows_v.at[pl.ds(p * C, C)], gsem.at[p]
            ).wait()

            @pl.when(s > 0)
            def _():
                for dblk in range(DB):
                    pltpu.make_async_copy(
                        obuf.at[dblk],
                        out_hbm.at[0, dblk, pl.ds(wid * CB, CB)],
                        osem,
                    ).wait()

            pbase = p * C

            @plsc.parallel_loop(0, DB * CB, unroll=2)
            def trans(u):
                dblk = u // CB
                bblk = lax.rem(u, CB)
                rbase = pbase + bblk * 128 + lane
                dbase = jnp.full((L,), dblk * 8, jnp.int32)
                for dsub in range(8):
                    dvec = dbase + dsub
                    for grp in range(128 // L):
                        rvec = rbase + grp * L
                        vals = plsc.load_gather(rows_v, [rvec, dvec])
                        obuf[dblk, bblk, dsub, pl.ds(grp * L, L)] = vals

            for dblk in range(DB):
                pltpu.async_copy(
                    obuf.at[dblk],
                    out_hbm.at[s, dblk, pl.ds(wid * CB, CB)],
                    osem,
                )
            return carry

        lax.fori_loop(0, S, body, 0)

        for dblk in range(DB):
            pltpu.make_async_copy(
                obuf.at[dblk],
                out_hbm.at[0, dblk, pl.ds(wid * CB, CB)],
                osem,
            ).wait()

    return gather_kernel


def kernel(inputs, embeddings):
    B, S = inputs.shape
    V, D = embeddings.shape
    idx_t = jnp.transpose(inputs).astype(jnp.int32)
    out5 = _build_gather(S, B, V, D, 512)(idx_t, embeddings)
    out = jnp.transpose(out5, (2, 4, 0, 1, 3)).reshape(B, S, D)
    return out
